# Initial kernel scaffold; baseline (speedup 1.0000x reference)
import jax
import jax.numpy as jnp
from jax.experimental import pallas as pl


def kernel(joint_features, data, k):
    raise NotImplementedError("write your pallas kernel here")


# speedup vs baseline: 54.4552x; 54.4552x over previous
import jax
import jax.numpy as jnp
from jax import lax
from jax.experimental import pallas as pl
from jax.experimental.pallas import tpu as pltpu

_J = 15
_K = 8
_GB = 256


def _masked_select_body(feats_ref, data_ref, out_ref):
    f = feats_ref[...]
    m = jnp.max(f, axis=-1, keepdims=True)
    e = jnp.exp(f - m)
    z = jnp.sum(e, axis=-1, keepdims=True)
    ent = jnp.log(z[..., 0]) - jnp.sum(e * (f - m), axis=-1) / z[..., 0]

    e_this = ent[:, :, None]
    e_other = ent[:, None, :]
    gb = ent.shape[0]
    j_this = lax.broadcasted_iota(jnp.int32, (gb, _J, _J), 1)
    j_other = lax.broadcasted_iota(jnp.int32, (gb, _J, _J), 2)
    beats = (e_other > e_this) | ((e_other == e_this) & (j_other < j_this))
    rank = jnp.sum(beats.astype(jnp.int32), axis=2)
    mask = (rank < _K).astype(jnp.float32)

    out_ref[...] = data_ref[...] * mask[:, :, None]


def kernel(joint_features, data, k):
    del k
    B, N, J, T, Dt = joint_features.shape
    C = data.shape[-1]
    D = 3
    G = B * N
    feats3 = joint_features.reshape(G, J, T * Dt)
    data3 = data.reshape(G, J, D * C)

    out3 = pl.pallas_call(
        _masked_select_body,
        grid=(G // _GB,),
        in_specs=[
            pl.BlockSpec((_GB, J, T * Dt), lambda i: (i, 0, 0)),
            pl.BlockSpec((_GB, J, D * C), lambda i: (i, 0, 0)),
        ],
        out_specs=pl.BlockSpec((_GB, J, D * C), lambda i: (i, 0, 0)),
        out_shape=jax.ShapeDtypeStruct((G, J, D * C), data.dtype),
        compiler_params=pltpu.CompilerParams(
            dimension_semantics=("parallel",),
        ),
    )(feats3, data3)

    return out3.reshape(B, N * J * D, C)


# speedup vs baseline: 70.0573x; 1.2865x over previous
import jax
import jax.numpy as jnp
from jax import lax
from jax.experimental import pallas as pl
from jax.experimental.pallas import tpu as pltpu

_J = 15
_K = 8
_GB = 256


def _masked_select_body(feats_ref, data_ref, out_ref):
    f = feats_ref[...]
    m = jnp.max(f, axis=-1, keepdims=True)
    e = jnp.exp(f - m)
    z = jnp.sum(e, axis=-1, keepdims=True)
    ent = jnp.log(z[..., 0]) - jnp.sum(e * (f - m), axis=-1) / z[..., 0]

    ent_t = ent.T
    gb = ent.shape[0]
    j_sub = lax.broadcasted_iota(jnp.int32, (_J, gb), 0)
    acc = jnp.zeros((_J, gb), jnp.float32)
    for r in range(1, _J):
        other = jnp.concatenate([ent_t[r:], ent_t[:r]], axis=0)
        wrap = j_sub >= (_J - r)
        beats = (other > ent_t) | ((other == ent_t) & wrap)
        acc = acc + beats.astype(jnp.float32)
    mask = (acc < float(_K)).astype(jnp.float32).T

    out_ref[...] = data_ref[...] * mask[:, :, None]


def kernel(joint_features, data, k):
    del k
    B, N, J, T, Dt = joint_features.shape
    C = data.shape[-1]
    D = 3
    G = B * N
    feats3 = joint_features.reshape(G, J, T * Dt)
    data3 = data.reshape(G, J, D * C)

    out3 = pl.pallas_call(
        _masked_select_body,
        grid=(G // _GB,),
        in_specs=[
            pl.BlockSpec((_GB, J, T * Dt), lambda i: (i, 0, 0)),
            pl.BlockSpec((_GB, J, D * C), lambda i: (i, 0, 0)),
        ],
        out_specs=pl.BlockSpec((_GB, J, D * C), lambda i: (i, 0, 0)),
        out_shape=jax.ShapeDtypeStruct((G, J, D * C), data.dtype),
        compiler_params=pltpu.CompilerParams(
            dimension_semantics=("parallel",),
        ),
    )(feats3, data3)

    return out3.reshape(B, N * J * D, C)


# speedup vs baseline: 531.0541x; 7.5803x over previous
import jax
import jax.numpy as jnp
from jax import lax
from jax.experimental import pallas as pl
from jax.experimental.pallas import tpu as pltpu

_J = 15
_K = 8
_D = 3


def _masked_select_body(f_ref, d_ref, g_ref, e_ref, o_ref):
    f = f_ref[0]
    n = f.shape[-1]
    m = jnp.max(f, axis=(1, 2), keepdims=True)
    ex = jnp.exp(f - m)
    z = jnp.sum(ex, axis=(1, 2), keepdims=True)
    s = jnp.sum(ex * (f - m), axis=(1, 2), keepdims=True)
    ent = (jnp.log(z) - s / z)[:, 0, 0, :]

    j_sub = lax.broadcasted_iota(jnp.int32, (_J, n), 0)
    acc = jnp.zeros((_J, n), jnp.float32)
    for r in range(1, _J):
        other = jnp.concatenate([ent[r:], ent[:r]], axis=0)
        wrap = j_sub >= (_J - r)
        beats = (other > ent) | ((other == ent) & wrap)
        acc = acc + beats.astype(jnp.float32)
    mask = (acc < float(_K)).astype(jnp.float32)

    t1 = jnp.dot(mask, g_ref[...], preferred_element_type=jnp.float32)
    lane_mask = jnp.sum(t1 * e_ref[...], axis=0, keepdims=True)
    o_ref[0] = d_ref[0] * lane_mask


def kernel(joint_features, data, k):
    del k
    B, N, J, T, Dt = joint_features.shape
    C = data.shape[-1]
    R = N * J * _D
    feats_t = jnp.transpose(joint_features, (0, 2, 4, 3, 1))
    data_t = jnp.transpose(data, (0, 2, 1))

    rr = jnp.arange(R, dtype=jnp.int32)[None, :]
    g_mat = (rr // (J * _D) == jnp.arange(N, dtype=jnp.int32)[:, None])
    e_mat = ((rr % (J * _D)) // _D == jnp.arange(J, dtype=jnp.int32)[:, None])

    out_t = pl.pallas_call(
        _masked_select_body,
        grid=(B,),
        in_specs=[
            pl.BlockSpec((1, J, _D, T, N), lambda i: (i, 0, 0, 0, 0)),
            pl.BlockSpec((1, C, R), lambda i: (i, 0, 0)),
            pl.BlockSpec((N, R), lambda i: (0, 0)),
            pl.BlockSpec((J, R), lambda i: (0, 0)),
        ],
        out_specs=pl.BlockSpec((1, C, R), lambda i: (i, 0, 0)),
        out_shape=jax.ShapeDtypeStruct((B, C, R), data.dtype),
        compiler_params=pltpu.CompilerParams(
            dimension_semantics=("arbitrary",),
        ),
    )(feats_t, data_t, g_mat.astype(jnp.float32), e_mat.astype(jnp.float32))

    return jnp.transpose(out_t, (0, 2, 1))


# speedup vs baseline: 547.2864x; 1.0306x over previous
import jax
import jax.numpy as jnp
from jax import lax
from jax.experimental import pallas as pl
from jax.experimental.pallas import tpu as pltpu

_J = 15
_K = 8
_D = 3


def _masked_select_body(f_ref, d_ref, o_ref, g_ref, e_ref):
    jd = _J * _D
    n_dim, r_dim = g_ref.shape

    @pl.when(pl.program_id(0) == 0)
    def _init_expansion_consts():
        rr_n = lax.broadcasted_iota(jnp.int32, (n_dim, r_dim), 1) // jd
        nn = lax.broadcasted_iota(jnp.int32, (n_dim, r_dim), 0)
        g_ref[...] = (rr_n == nn).astype(jnp.float32)
        rr_j = lax.broadcasted_iota(jnp.int32, (_J, r_dim), 1) % jd // _D
        jj = lax.broadcasted_iota(jnp.int32, (_J, r_dim), 0)
        e_ref[...] = (rr_j == jj).astype(jnp.float32)

    f = f_ref[0]
    n = f.shape[-1]
    m = jnp.max(f, axis=(1, 2), keepdims=True)
    ex = jnp.exp(f - m)
    z = jnp.sum(ex, axis=(1, 2), keepdims=True)
    s = jnp.sum(ex * (f - m), axis=(1, 2), keepdims=True)
    ent = (jnp.log(z) - s / z)[:, 0, 0, :]

    j_sub = lax.broadcasted_iota(jnp.int32, (_J, n), 0)
    acc = jnp.zeros((_J, n), jnp.float32)
    for r in range(1, _J):
        other = jnp.concatenate([ent[r:], ent[:r]], axis=0)
        wrap = j_sub >= (_J - r)
        beats = (other > ent) | ((other == ent) & wrap)
        acc = acc + beats.astype(jnp.float32)
    mask = (acc < float(_K)).astype(jnp.float32)

    t1 = jnp.dot(mask, g_ref[...], preferred_element_type=jnp.float32)
    lane_mask = jnp.sum(t1 * e_ref[...], axis=0, keepdims=True)
    o_ref[0] = d_ref[0] * lane_mask


def kernel(joint_features, data, k):
    del k
    B, N, J, T, Dt = joint_features.shape
    C = data.shape[-1]
    R = N * J * _D
    feats_t = jnp.transpose(joint_features, (0, 2, 4, 3, 1))
    data_t = jnp.transpose(data, (0, 2, 1))

    out_t = pl.pallas_call(
        _masked_select_body,
        grid=(B,),
        in_specs=[
            pl.BlockSpec((1, J, _D, T, N), lambda i: (i, 0, 0, 0, 0)),
            pl.BlockSpec((1, C, R), lambda i: (i, 0, 0)),
        ],
        out_specs=pl.BlockSpec((1, C, R), lambda i: (i, 0, 0)),
        out_shape=jax.ShapeDtypeStruct((B, C, R), data.dtype),
        scratch_shapes=[
            pltpu.VMEM((N, R), jnp.float32),
            pltpu.VMEM((J, R), jnp.float32),
        ],
        compiler_params=pltpu.CompilerParams(
            dimension_semantics=("arbitrary",),
        ),
    )(feats_t, data_t)

    return jnp.transpose(out_t, (0, 2, 1))
